# baseline (speedup 1.0000x reference)
import jax
import jax.numpy as jnp
from jax import lax
from jax.experimental import pallas as pl
from jax.experimental.pallas import tpu as pltpu
from jax.experimental.pallas import tpu_sc as plsc

B, H, W, C = 16, 384, 384, 96
HW = H * W
NC, NS, L = 2, 16, 16
NW = NC * NS
PPW = B * HW // NW
N = 96
CHUNKS = PPW // N
G = N // L
CB = 8
NCB = N * CB
Wf = float(W)
Hf = float(H)

_DNUMS = lax.GatherDimensionNumbers(
    offset_dims=(), collapsed_slice_dims=(0,), start_index_map=(0,))


def _splat(v, l):
    idx = jnp.full((L, 1), l, jnp.int32)
    return lax.gather(v, idx, _DNUMS, (1,),
                      mode=lax.GatherScatterMode.PROMISE_IN_BOUNDS)


def _gen_and_fire(c, pix0, xf_hbm, yf_hbm, nbr_hbm,
                  cbx, cby, idx, ws, gbuf, sem):
    @pl.when(lax.rem(c, CB) == 0)
    def _():
        off = pix0 + c * N
        pltpu.sync_copy(xf_hbm.at[pl.ds(off, NCB)], cbx)
        pltpu.sync_copy(yf_hbm.at[pl.ds(off, NCB)], cby)

    pos = lax.rem(c, CB) * N
    base = lax.div(pix0 + c * N, HW) * HW
    wA, wB, wC, wD = ws
    for g in range(G):
        s = pl.ds(g * L, L)
        Xf = cbx[pl.ds(pos + g * L, L)]
        Yf = cby[pl.ds(pos + g * L, L)]
        Xp = (Xf + 1.0) / 2.0 * Wf
        Yp = (Yf + 1.0) / 2.0 * Hf
        x0t = Xp.astype(jnp.int32)
        x0 = jnp.where(x0t.astype(jnp.float32) > Xp, x0t - 1, x0t)
        y0t = Yp.astype(jnp.int32)
        y0 = jnp.where(y0t.astype(jnp.float32) > Yp, y0t - 1, y0t)
        m = ((Xp >= 0.0) & (Xp < Wf - 1.0)
             & (Yp >= 0.0) & (Yp < Hf - 1.0))
        x0c = jnp.minimum(jnp.maximum(x0, 0), W - 2)
        y0c = jnp.minimum(jnp.maximum(y0, 0), H - 2)
        x0f = x0c.astype(jnp.float32)
        y0f = y0c.astype(jnp.float32)
        x1f = x0f + 1.0
        y1f = y0f + 1.0
        zero = jnp.zeros((L,), jnp.float32)
        wa = jnp.where(m, (x1f - Xp) * (y1f - Yp), zero)
        wb = jnp.where(m, (x1f - Xp) * (Yp - y0f), zero)
        wc = jnp.where(m, (Xp - x0f) * (y1f - Yp), zero)
        wd = jnp.where(m, (Xp - x0f) * (Yp - y0f), zero)
        idx[s] = base + y0c * W + x0c
        wA[s] = wa
        wB[s] = wb
        wC[s] = wc
        wD[s] = wd
    pltpu.async_copy(nbr_hbm.at[idx], gbuf, sem)


def _combine(ws, gbuf, ob):
    wA, wB, wC, wD = ws

    def gbody(g, carry):
        s = pl.ds(g * L, L)
        wa16 = wA[s]
        wb16 = wB[s]
        wc16 = wC[s]
        wd16 = wD[s]
        for l in range(L):
            i = g * L + l
            wal = wa16[l]
            wbl = wb16[l]
            wcl = wc16[l]
            wdl = wd16[l]
            for j in range(C // L):
                av = gbuf[i, pl.ds(j * L, L)]
                cv = gbuf[i, pl.ds(C + j * L, L)]
                bv = gbuf[i, pl.ds(2 * C + j * L, L)]
                dv = gbuf[i, pl.ds(3 * C + j * L, L)]
                o = av * wal + bv * wbl
                o = o + cv * wcl
                o = o + dv * wdl
                ob[pl.ds(i * C + j * L, L)] = o
        return carry

    lax.fori_loop(0, G, gbody, 0)


def _body(nbr_hbm, xf_hbm, yf_hbm, out_hbm,
          cbx, cby,
          idx0, idx1,
          wA0, wB0, wC0, wD0, wA1, wB1, wC1, wD1,
          g0, g1, ob0, ob1,
          sem_g0, sem_g1, sem_o0, sem_o1):
    wid = lax.axis_index("c") * NS + lax.axis_index("s")
    pix0 = wid * PPW

    idxs = (idx0, idx1)
    ws = ((wA0, wB0, wC0, wD0), (wA1, wB1, wC1, wD1))
    gbufs = (g0, g1)
    obs = (ob0, ob1)
    sem_gs = (sem_g0, sem_g1)
    sem_os = (sem_o0, sem_o1)

    _gen_and_fire(0, pix0, xf_hbm, yf_hbm, nbr_hbm,
                  cbx, cby, idxs[0], ws[0], gbufs[0], sem_gs[0])

    def outer(i2, carry):
        for d in (0, 1):
            c = i2 * 2 + d
            nd = 1 - d

            @pl.when(c + 1 < CHUNKS)
            def _():
                _gen_and_fire(c + 1, pix0, xf_hbm, yf_hbm, nbr_hbm,
                              cbx, cby, idxs[nd], ws[nd], gbufs[nd],
                              sem_gs[nd])

            pltpu.make_async_copy(nbr_hbm.at[idxs[d]], gbufs[d],
                                  sem_gs[d]).wait()

            @pl.when(c >= 2)
            def _():
                prev = (pix0 + (c - 2) * N) * C
                pltpu.make_async_copy(
                    obs[d], out_hbm.at[pl.ds(prev, N * C)], sem_os[d]).wait()

            _combine(ws[d], gbufs[d], obs[d])
            cur = (pix0 + c * N) * C
            pltpu.async_copy(obs[d], out_hbm.at[pl.ds(cur, N * C)], sem_os[d])
        return carry

    lax.fori_loop(0, CHUNKS // 2, outer, 0)

    for d in (0, 1):
        last = (pix0 + (CHUNKS - 2 + d) * N) * C
        pltpu.make_async_copy(
            obs[d], out_hbm.at[pl.ds(last, N * C)], sem_os[d]).wait()


@jax.jit
def kernel(im, thetas):
    X, Y = jnp.meshgrid(jnp.linspace(-1.0, 1.0, W), jnp.linspace(-1.0, 1.0, H))
    flat_coords = jnp.concatenate(
        [X.reshape(1, -1), Y.reshape(1, -1),
         jnp.ones((1, H * W), dtype=jnp.float32)], axis=0)
    th = thetas.reshape(-1, 2, 3)
    new_flat = jnp.matmul(th, jnp.broadcast_to(flat_coords[None, :, :],
                                               (B, 3, H * W)))
    Xall = new_flat[:, 0, :].reshape(-1)
    Yall = new_flat[:, 1, :].reshape(-1)

    mesh = plsc.VectorSubcoreMesh(core_axis_name="c", subcore_axis_name="s",
                                  num_cores=NC, num_subcores=NS)
    scratch = (
        [pltpu.VMEM((NCB,), jnp.float32) for _ in range(2)]
        + [pltpu.VMEM((N,), jnp.int32) for _ in range(2)]
        + [pltpu.VMEM((N,), jnp.float32) for _ in range(8)]
        + [pltpu.VMEM((N, 4 * C), jnp.float32) for _ in range(2)]
        + [pltpu.VMEM((N * C,), jnp.float32) for _ in range(2)]
        + [pltpu.SemaphoreType.DMA for _ in range(4)]
    )
    sc_call = pl.kernel(
        _body,
        out_type=jax.ShapeDtypeStruct((B * HW * C,), jnp.float32),
        mesh=mesh,
        scratch_types=scratch,
    )

    imf = im.reshape(B * HW, C)
    nbr = jnp.concatenate(
        [imf,
         jnp.roll(imf, -1, axis=0),
         jnp.roll(imf, -W, axis=0),
         jnp.roll(imf, -(W + 1), axis=0)], axis=1)
    return sc_call(nbr, Xall, Yall).reshape(B, H, W, C)


# speedup vs baseline: 1.1769x; 1.1769x over previous
import jax
import jax.numpy as jnp
from jax import lax
from jax.experimental import pallas as pl
from jax.experimental.pallas import tpu as pltpu
from jax.experimental.pallas import tpu_sc as plsc

B, H, W, C = 16, 384, 384, 96
HW = H * W
NC, NS, L = 2, 16, 16
NW = NC * NS
PPW = B * HW // NW
N = 96
CHUNKS = PPW // N
G = N // L
CB = 8
NCB = N * CB
Wf = float(W)
Hf = float(H)

_DNUMS = lax.GatherDimensionNumbers(
    offset_dims=(), collapsed_slice_dims=(0,), start_index_map=(0,))


def _splat(v, l):
    idx = jnp.full((L, 1), l, jnp.int32)
    return lax.gather(v, idx, _DNUMS, (1,),
                      mode=lax.GatherScatterMode.PROMISE_IN_BOUNDS)


def _gen_and_fire(c, pix0, xf_hbm, yf_hbm, nbr_hbm,
                  cbx, cby, idx, ws, gbuf, sem, livef):
    @pl.when(lax.rem(c, CB) == 0)
    def _():
        off = pix0 + c * N
        pltpu.sync_copy(xf_hbm.at[pl.ds(off, NCB)], cbx)
        pltpu.sync_copy(yf_hbm.at[pl.ds(off, NCB)], cby)

    pos = lax.rem(c, CB) * N
    base = lax.div(pix0 + c * N, HW) * HW

    @pl.when(livef != 0)
    def _():
        wA, wB, wC, wD = ws
        for g in range(G):
            s = pl.ds(g * L, L)
            Xf = cbx[pl.ds(pos + g * L, L)]
            Yf = cby[pl.ds(pos + g * L, L)]
            Xp = (Xf + 1.0) / 2.0 * Wf
            Yp = (Yf + 1.0) / 2.0 * Hf
            x0t = Xp.astype(jnp.int32)
            x0 = jnp.where(x0t.astype(jnp.float32) > Xp, x0t - 1, x0t)
            y0t = Yp.astype(jnp.int32)
            y0 = jnp.where(y0t.astype(jnp.float32) > Yp, y0t - 1, y0t)
            m = ((Xp >= 0.0) & (Xp < Wf - 1.0)
                 & (Yp >= 0.0) & (Yp < Hf - 1.0))
            x0c = jnp.minimum(jnp.maximum(x0, 0), W - 2)
            y0c = jnp.minimum(jnp.maximum(y0, 0), H - 2)
            x0f = x0c.astype(jnp.float32)
            y0f = y0c.astype(jnp.float32)
            x1f = x0f + 1.0
            y1f = y0f + 1.0
            zero = jnp.zeros((L,), jnp.float32)
            wa = jnp.where(m, (x1f - Xp) * (y1f - Yp), zero)
            wb = jnp.where(m, (x1f - Xp) * (Yp - y0f), zero)
            wc = jnp.where(m, (Xp - x0f) * (y1f - Yp), zero)
            wd = jnp.where(m, (Xp - x0f) * (Yp - y0f), zero)
            idx[s] = base + y0c * W + x0c
            wA[s] = wa
            wB[s] = wb
            wC[s] = wc
            wD[s] = wd
        pltpu.async_copy(nbr_hbm.at[idx], gbuf, sem)


def _combine(ws, gbuf, ob):
    wA, wB, wC, wD = ws

    def gbody(g, carry):
        s = pl.ds(g * L, L)
        wa16 = wA[s]
        wb16 = wB[s]
        wc16 = wC[s]
        wd16 = wD[s]
        for l in range(L):
            i = g * L + l
            wal = wa16[l]
            wbl = wb16[l]
            wcl = wc16[l]
            wdl = wd16[l]
            for j in range(C // L):
                av = gbuf[i, pl.ds(j * L, L)]
                cv = gbuf[i, pl.ds(C + j * L, L)]
                bv = gbuf[i, pl.ds(2 * C + j * L, L)]
                dv = gbuf[i, pl.ds(3 * C + j * L, L)]
                o = av * wal + bv * wbl
                o = o + cv * wcl
                o = o + dv * wdl
                ob[pl.ds(i * C + j * L, L)] = o
        return carry

    lax.fori_loop(0, G, gbody, 0)


def _zero(ob):
    zero = jnp.zeros((L,), jnp.float32)

    def zbody(k, carry):
        ob[pl.ds(k * L, L)] = zero
        return carry

    lax.fori_loop(0, N * C // L, zbody, 0)


def _body(nbr_hbm, xf_hbm, yf_hbm, flg_hbm, out_hbm,
          cbx, cby,
          idx0, idx1,
          wA0, wB0, wC0, wD0, wA1, wB1, wC1, wD1,
          g0, g1, ob0, ob1,
          fvmem, dirty,
          sem_g0, sem_g1, sem_o0, sem_o1):
    wid = lax.axis_index("c") * NS + lax.axis_index("s")
    pix0 = wid * PPW

    idxs = (idx0, idx1)
    ws = ((wA0, wB0, wC0, wD0), (wA1, wB1, wC1, wD1))
    gbufs = (g0, g1)
    obs = (ob0, ob1)
    sem_gs = (sem_g0, sem_g1)
    sem_os = (sem_o0, sem_o1)

    pltpu.sync_copy(flg_hbm.at[pl.ds(wid * CHUNKS, CHUNKS + L)], fvmem)

    dirty[0] = 1
    dirty[1] = 1

    fv0 = fvmem[pl.ds(0, L)]
    _gen_and_fire(0, pix0, xf_hbm, yf_hbm, nbr_hbm,
                  cbx, cby, idxs[0], ws[0], gbufs[0], sem_gs[0], fv0[0])

    def outer(i2, carry):
        fvv = fvmem[pl.ds(i2 * 2, L)]
        for d in (0, 1):
            c = i2 * 2 + d
            nd = 1 - d
            live = fvv[d] != 0

            @pl.when(c + 1 < CHUNKS)
            def _():
                _gen_and_fire(c + 1, pix0, xf_hbm, yf_hbm, nbr_hbm,
                              cbx, cby, idxs[nd], ws[nd], gbufs[nd],
                              sem_gs[nd], fvv[d + 1])

            @pl.when(live)
            def _():
                pltpu.make_async_copy(nbr_hbm.at[idxs[d]], gbufs[d],
                                      sem_gs[d]).wait()

            @pl.when(c >= 2)
            def _():
                prev = (pix0 + (c - 2) * N) * C
                pltpu.make_async_copy(
                    obs[d], out_hbm.at[pl.ds(prev, N * C)], sem_os[d]).wait()

            @pl.when(live)
            def _():
                _combine(ws[d], gbufs[d], obs[d])

            @pl.when(jnp.logical_and(jnp.logical_not(live), dirty[d] != 0))
            def _():
                _zero(obs[d])

            dirty[d] = fvv[d]
            cur = (pix0 + c * N) * C
            pltpu.async_copy(obs[d], out_hbm.at[pl.ds(cur, N * C)], sem_os[d])
        return carry

    lax.fori_loop(0, CHUNKS // 2, outer, 0)

    for d in (0, 1):
        last = (pix0 + (CHUNKS - 2 + d) * N) * C
        pltpu.make_async_copy(
            obs[d], out_hbm.at[pl.ds(last, N * C)], sem_os[d]).wait()


@jax.jit
def kernel(im, thetas):
    X, Y = jnp.meshgrid(jnp.linspace(-1.0, 1.0, W), jnp.linspace(-1.0, 1.0, H))
    flat_coords = jnp.concatenate(
        [X.reshape(1, -1), Y.reshape(1, -1),
         jnp.ones((1, H * W), dtype=jnp.float32)], axis=0)
    th = thetas.reshape(-1, 2, 3)
    new_flat = jnp.matmul(th, jnp.broadcast_to(flat_coords[None, :, :],
                                               (B, 3, H * W)))
    Xall = new_flat[:, 0, :].reshape(-1)
    Yall = new_flat[:, 1, :].reshape(-1)

    mesh = plsc.VectorSubcoreMesh(core_axis_name="c", subcore_axis_name="s",
                                  num_cores=NC, num_subcores=NS)
    scratch = (
        [pltpu.VMEM((NCB,), jnp.float32) for _ in range(2)]
        + [pltpu.VMEM((N,), jnp.int32) for _ in range(2)]
        + [pltpu.VMEM((N,), jnp.float32) for _ in range(8)]
        + [pltpu.VMEM((N, 4 * C), jnp.float32) for _ in range(2)]
        + [pltpu.VMEM((N * C,), jnp.float32) for _ in range(2)]
        + [pltpu.VMEM((CHUNKS + L,), jnp.int32),
           pltpu.SMEM((2,), jnp.int32)]
        + [pltpu.SemaphoreType.DMA for _ in range(4)]
    )
    sc_call = pl.kernel(
        _body,
        out_type=jax.ShapeDtypeStruct((B * HW * C,), jnp.float32),
        mesh=mesh,
        scratch_types=scratch,
    )

    imf = im.reshape(B * HW, C)
    nbr = jnp.concatenate(
        [imf,
         jnp.roll(imf, -1, axis=0),
         jnp.roll(imf, -W, axis=0),
         jnp.roll(imf, -(W + 1), axis=0)], axis=1)

    Xp_h = (Xall + 1.0) / 2.0 * Wf
    Yp_h = (Yall + 1.0) / 2.0 * Hf
    m_h = ((Xp_h >= 0.0) & (Xp_h < Wf - 1.0)
           & (Yp_h >= 0.0) & (Yp_h < Hf - 1.0))
    chunk_live = m_h.reshape(-1, N).any(axis=1).astype(jnp.int32)
    chunk_live = jnp.concatenate(
        [chunk_live, jnp.zeros((L,), jnp.int32)])

    return sc_call(nbr, Xall, Yall, chunk_live).reshape(B, H, W, C)


# speedup vs baseline: 1.3100x; 1.1131x over previous
import jax
import jax.numpy as jnp
from jax import lax
from jax.experimental import pallas as pl
from jax.experimental.pallas import tpu as pltpu
from jax.experimental.pallas import tpu_sc as plsc

B, H, W, C = 16, 384, 384, 96
HW = H * W
NC, NS, L = 2, 16, 16
NW = NC * NS
PPW = B * HW // NW
N = 96
CHUNKS = PPW // N
G = N // L
CB = 8
NCB = N * CB
Wf = float(W)
Hf = float(H)

_DNUMS = lax.GatherDimensionNumbers(
    offset_dims=(), collapsed_slice_dims=(0,), start_index_map=(0,))


def _splat(v, l):
    idx = jnp.full((L, 1), l, jnp.int32)
    return lax.gather(v, idx, _DNUMS, (1,),
                      mode=lax.GatherScatterMode.PROMISE_IN_BOUNDS)


def _gpix(wid, c):
    return (wid + (c // CB) * NW) * NCB + (c % CB) * N


def _gen_and_fire(c, wid, xf_hbm, yf_hbm, nbr_hbm,
                  cbx, cby, idx, ws, gbuf, sem, livef):
    @pl.when(lax.rem(c, CB) == 0)
    def _():
        off = (wid + lax.div(c, CB) * NW) * NCB
        pltpu.sync_copy(xf_hbm.at[pl.ds(off, NCB)], cbx)
        pltpu.sync_copy(yf_hbm.at[pl.ds(off, NCB)], cby)

    pos = lax.rem(c, CB) * N
    base = lax.div(_gpix(wid, c), HW) * HW

    @pl.when(livef != 0)
    def _():
        wA, wB, wC, wD = ws
        for g in range(G):
            s = pl.ds(g * L, L)
            Xf = cbx[pl.ds(pos + g * L, L)]
            Yf = cby[pl.ds(pos + g * L, L)]
            Xp = (Xf + 1.0) / 2.0 * Wf
            Yp = (Yf + 1.0) / 2.0 * Hf
            x0t = Xp.astype(jnp.int32)
            x0 = jnp.where(x0t.astype(jnp.float32) > Xp, x0t - 1, x0t)
            y0t = Yp.astype(jnp.int32)
            y0 = jnp.where(y0t.astype(jnp.float32) > Yp, y0t - 1, y0t)
            m = ((Xp >= 0.0) & (Xp < Wf - 1.0)
                 & (Yp >= 0.0) & (Yp < Hf - 1.0))
            x0c = jnp.minimum(jnp.maximum(x0, 0), W - 2)
            y0c = jnp.minimum(jnp.maximum(y0, 0), H - 2)
            x0f = x0c.astype(jnp.float32)
            y0f = y0c.astype(jnp.float32)
            x1f = x0f + 1.0
            y1f = y0f + 1.0
            zero = jnp.zeros((L,), jnp.float32)
            wa = jnp.where(m, (x1f - Xp) * (y1f - Yp), zero)
            wb = jnp.where(m, (x1f - Xp) * (Yp - y0f), zero)
            wc = jnp.where(m, (Xp - x0f) * (y1f - Yp), zero)
            wd = jnp.where(m, (Xp - x0f) * (Yp - y0f), zero)
            idx[s] = base + y0c * W + x0c
            wA[s] = wa
            wB[s] = wb
            wC[s] = wc
            wD[s] = wd
        pltpu.async_copy(nbr_hbm.at[idx], gbuf, sem)


def _combine(ws, gbuf, ob):
    wA, wB, wC, wD = ws

    def gbody(g, carry):
        s = pl.ds(g * L, L)
        wa16 = wA[s]
        wb16 = wB[s]
        wc16 = wC[s]
        wd16 = wD[s]
        for l in range(L):
            i = g * L + l
            wal = wa16[l]
            wbl = wb16[l]
            wcl = wc16[l]
            wdl = wd16[l]
            for j in range(C // L):
                av = gbuf[i, pl.ds(j * L, L)]
                cv = gbuf[i, pl.ds(C + j * L, L)]
                bv = gbuf[i, pl.ds(2 * C + j * L, L)]
                dv = gbuf[i, pl.ds(3 * C + j * L, L)]
                o = av * wal + bv * wbl
                o = o + cv * wcl
                o = o + dv * wdl
                ob[pl.ds(i * C + j * L, L)] = o
        return carry

    lax.fori_loop(0, G, gbody, 0)


def _zero(ob):
    zero = jnp.zeros((L,), jnp.float32)

    def zbody(k, carry):
        ob[pl.ds(k * L, L)] = zero
        return carry

    lax.fori_loop(0, N * C // L, zbody, 0)


def _body(nbr_hbm, xf_hbm, yf_hbm, flg_hbm, out_hbm,
          cbx, cby,
          idx0, idx1,
          wA0, wB0, wC0, wD0, wA1, wB1, wC1, wD1,
          g0, g1, ob0, ob1,
          fvmem, dirty,
          sem_g0, sem_g1, sem_o0, sem_o1):
    wid = lax.axis_index("c") * NS + lax.axis_index("s")

    idxs = (idx0, idx1)
    ws = ((wA0, wB0, wC0, wD0), (wA1, wB1, wC1, wD1))
    gbufs = (g0, g1)
    obs = (ob0, ob1)
    sem_gs = (sem_g0, sem_g1)
    sem_os = (sem_o0, sem_o1)

    pltpu.sync_copy(flg_hbm.at[pl.ds(wid * CHUNKS, CHUNKS + L)], fvmem)

    dirty[0] = 1
    dirty[1] = 1

    fv0 = fvmem[pl.ds(0, L)]
    _gen_and_fire(0, wid, xf_hbm, yf_hbm, nbr_hbm,
                  cbx, cby, idxs[0], ws[0], gbufs[0], sem_gs[0], fv0[0])

    def outer(i2, carry):
        fvv = fvmem[pl.ds(i2 * 2, L)]
        for d in (0, 1):
            c = i2 * 2 + d
            nd = 1 - d
            live = fvv[d] != 0

            @pl.when(c + 1 < CHUNKS)
            def _():
                _gen_and_fire(c + 1, wid, xf_hbm, yf_hbm, nbr_hbm,
                              cbx, cby, idxs[nd], ws[nd], gbufs[nd],
                              sem_gs[nd], fvv[d + 1])

            @pl.when(live)
            def _():
                pltpu.make_async_copy(nbr_hbm.at[idxs[d]], gbufs[d],
                                      sem_gs[d]).wait()

            @pl.when(c >= 2)
            def _():
                prev = _gpix(wid, c - 2) * C
                pltpu.make_async_copy(
                    obs[d], out_hbm.at[pl.ds(prev, N * C)], sem_os[d]).wait()

            @pl.when(live)
            def _():
                _combine(ws[d], gbufs[d], obs[d])

            @pl.when(jnp.logical_and(jnp.logical_not(live), dirty[d] != 0))
            def _():
                _zero(obs[d])

            dirty[d] = fvv[d]
            cur = _gpix(wid, c) * C
            pltpu.async_copy(obs[d], out_hbm.at[pl.ds(cur, N * C)], sem_os[d])
        return carry

    lax.fori_loop(0, CHUNKS // 2, outer, 0)

    for d in (0, 1):
        last = _gpix(wid, CHUNKS - 2 + d) * C
        pltpu.make_async_copy(
            obs[d], out_hbm.at[pl.ds(last, N * C)], sem_os[d]).wait()


@jax.jit
def kernel(im, thetas):
    X, Y = jnp.meshgrid(jnp.linspace(-1.0, 1.0, W), jnp.linspace(-1.0, 1.0, H))
    flat_coords = jnp.concatenate(
        [X.reshape(1, -1), Y.reshape(1, -1),
         jnp.ones((1, H * W), dtype=jnp.float32)], axis=0)
    th = thetas.reshape(-1, 2, 3)
    new_flat = jnp.matmul(th, jnp.broadcast_to(flat_coords[None, :, :],
                                               (B, 3, H * W)))
    Xall = new_flat[:, 0, :].reshape(-1)
    Yall = new_flat[:, 1, :].reshape(-1)

    mesh = plsc.VectorSubcoreMesh(core_axis_name="c", subcore_axis_name="s",
                                  num_cores=NC, num_subcores=NS)
    scratch = (
        [pltpu.VMEM((NCB,), jnp.float32) for _ in range(2)]
        + [pltpu.VMEM((N,), jnp.int32) for _ in range(2)]
        + [pltpu.VMEM((N,), jnp.float32) for _ in range(8)]
        + [pltpu.VMEM((N, 4 * C), jnp.float32) for _ in range(2)]
        + [pltpu.VMEM((N * C,), jnp.float32) for _ in range(2)]
        + [pltpu.VMEM((CHUNKS + L,), jnp.int32),
           pltpu.SMEM((2,), jnp.int32)]
        + [pltpu.SemaphoreType.DMA for _ in range(4)]
    )
    sc_call = pl.kernel(
        _body,
        out_type=jax.ShapeDtypeStruct((B * HW * C,), jnp.float32),
        mesh=mesh,
        scratch_types=scratch,
    )

    imf = im.reshape(B * HW, C)
    nbr = jnp.concatenate(
        [imf,
         jnp.roll(imf, -1, axis=0),
         jnp.roll(imf, -W, axis=0),
         jnp.roll(imf, -(W + 1), axis=0)], axis=1)

    Xp_h = (Xall + 1.0) / 2.0 * Wf
    Yp_h = (Yall + 1.0) / 2.0 * Hf
    m_h = ((Xp_h >= 0.0) & (Xp_h < Wf - 1.0)
           & (Yp_h >= 0.0) & (Yp_h < Hf - 1.0))
    chunk_live = m_h.reshape(-1, N).any(axis=1).astype(jnp.int32)
    chunk_live = (chunk_live.reshape(CHUNKS // CB, NW, CB)
                  .transpose(1, 0, 2).reshape(-1))
    chunk_live = jnp.concatenate(
        [chunk_live, jnp.zeros((L,), jnp.int32)])

    return sc_call(nbr, Xall, Yall, chunk_live).reshape(B, H, W, C)
